# Initial kernel scaffold; baseline (speedup 1.0000x reference)
import functools

import jax
import jax.numpy as jnp
from jax import lax
from jax.experimental import pallas as pl
from jax.experimental.pallas import tpu as pltpu
from jax.experimental.pallas import tpu_sc as plsc

N = 10000
D = 128
H = 128
NUM_GRAPHS = 64

NC = 2
NS = 16
NW = NC * NS
CHUNK = 128
ACC = 10016
ZPS = ACC // NS
WPS = N // NS

_mesh = lambda: plsc.VectorSubcoreMesh(core_axis_name="c", subcore_axis_name="s")


def _sc_degree(dst_p, zeros16, ones16, cpt):
    @functools.partial(
        pl.kernel,
        out_type=jax.ShapeDtypeStruct((NC, N, 16), jnp.float32),
        mesh=_mesh(),
        scratch_types=[
            pltpu.VMEM((cpt, CHUNK), jnp.int32),
            pltpu.VMEM((CHUNK, 16), jnp.float32),
            pltpu.VMEM_SHARED((ACC, 16), jnp.float32),
        ],
    )
    def k(dst_hbm, z_hbm, ones_hbm, out_hbm, dst_v, ones_v, shared):
        c = lax.axis_index("c")
        s = lax.axis_index("s")
        w = c * NS + s
        pltpu.sync_copy(dst_hbm.at[w], dst_v)
        pltpu.sync_copy(ones_hbm, ones_v)
        pltpu.sync_copy(z_hbm.at[pl.ds(s * ZPS, ZPS)], shared.at[pl.ds(s * ZPS, ZPS)])
        plsc.subcore_barrier()

        @pl.loop(0, cpt)
        def _(i):
            pltpu.sync_copy(ones_v, shared.at[dst_v.at[i]], add=True)

        plsc.subcore_barrier()
        pltpu.sync_copy(shared.at[pl.ds(s * WPS, WPS)],
                        out_hbm.at[c, pl.ds(s * WPS, WPS)])

    return k(dst_p, zeros16, ones16)


def _sc_aggregate(y, src_p, dst_p, zeros128, cpt):
    @functools.partial(
        pl.kernel,
        out_type=jax.ShapeDtypeStruct((NC, N, H), jnp.float32),
        mesh=_mesh(),
        scratch_types=[
            pltpu.VMEM((cpt, CHUNK), jnp.int32),
            pltpu.VMEM((cpt, CHUNK), jnp.int32),
            pltpu.VMEM((CHUNK, H), jnp.float32),
            pltpu.VMEM_SHARED((ACC, H), jnp.float32),
        ],
    )
    def k(y_hbm, src_hbm, dst_hbm, z_hbm, out_hbm, src_v, dst_v, rows_v, shared):
        c = lax.axis_index("c")
        s = lax.axis_index("s")
        w = c * NS + s
        pltpu.sync_copy(src_hbm.at[w], src_v)
        pltpu.sync_copy(dst_hbm.at[w], dst_v)
        pltpu.sync_copy(z_hbm.at[pl.ds(s * ZPS, ZPS)], shared.at[pl.ds(s * ZPS, ZPS)])
        plsc.subcore_barrier()

        @pl.loop(0, cpt)
        def _(i):
            pltpu.sync_copy(y_hbm.at[src_v.at[i]], rows_v)
            pltpu.sync_copy(rows_v, shared.at[dst_v.at[i]], add=True)

        plsc.subcore_barrier()
        pltpu.sync_copy(shared.at[pl.ds(s * WPS, WPS)],
                        out_hbm.at[c, pl.ds(s * WPS, WPS)])

    return k(y, src_p, dst_p, zeros128)


def _tc_dis_y1(degp, x, W1):
    def body(deg_ref, x_ref, w_ref, dis_ref, y_ref):
        d = deg_ref[0] + deg_ref[1]
        dis = lax.rsqrt(d[:, 0:1] + 1.0)
        dis_ref[...] = dis
        y_ref[...] = jnp.dot(x_ref[...], w_ref[...],
                             preferred_element_type=jnp.float32) * dis

    return pl.pallas_call(
        body,
        out_shape=(jax.ShapeDtypeStruct((N, 1), jnp.float32),
                   jax.ShapeDtypeStruct((N, H), jnp.float32)),
    )(degp, x, W1)


def _tc_layer(acc, y, dis, b, W_next):
    def body(acc_ref, y_ref, dis_ref, b_ref, w_ref, out_ref):
        dis = dis_ref[...]
        h = jnp.maximum((acc_ref[0] + acc_ref[1] + y_ref[...]) * dis + b_ref[...],
                        0.0)
        out_ref[...] = jnp.dot(h, w_ref[...],
                               preferred_element_type=jnp.float32) * dis

    return pl.pallas_call(
        body,
        out_shape=jax.ShapeDtypeStruct((N, H), jnp.float32),
    )(acc, y, dis, b, W_next)


def _tc_final(acc, y, dis, b, batch2d, lin_W, lin_b2d):
    def body(acc_ref, y_ref, dis_ref, b_ref, batch_ref, lw_ref, lb_ref, out_ref):
        dis = dis_ref[...]
        h = jnp.maximum((acc_ref[0] + acc_ref[1] + y_ref[...]) * dis + b_ref[...],
                        0.0)
        g = lax.broadcasted_iota(jnp.int32, (1, NUM_GRAPHS), 1)
        m = (batch_ref[...] == g).astype(jnp.float32)
        sums = lax.dot_general(m, h, (((0,), (0,)), ((), ())),
                               preferred_element_type=jnp.float32)
        cnts = jnp.sum(m, axis=0)[:, None]
        pooled = sums / jnp.maximum(cnts, 1.0)
        out_ref[...] = jnp.dot(pooled, lw_ref[...],
                               preferred_element_type=jnp.float32) + lb_ref[...]

    return pl.pallas_call(
        body,
        out_shape=jax.ShapeDtypeStruct((NUM_GRAPHS, 1), jnp.float32),
    )(acc, y, dis, b, batch2d, lin_W, lin_b2d)


def kernel(x, edge_index, batch, W1, b1, W2, b2, W3, b3, lin_W, lin_b):
    E = edge_index.shape[1]
    epad = ((E + NW * CHUNK - 1) // (NW * CHUNK)) * (NW * CHUNK)
    cpt = epad // (NW * CHUNK)

    src = edge_index[0].astype(jnp.int32)
    dst = edge_index[1].astype(jnp.int32)
    src_p = jnp.concatenate(
        [src, jnp.zeros((epad - E,), jnp.int32)]).reshape(NW, cpt, CHUNK)
    dst_p = jnp.concatenate(
        [dst, jnp.full((epad - E,), N, jnp.int32)]).reshape(NW, cpt, CHUNK)

    zeros16 = jnp.zeros((ACC, 16), jnp.float32)
    ones16 = jnp.ones((CHUNK, 16), jnp.float32)
    zeros128 = jnp.zeros((ACC, H), jnp.float32)
    batch2d = batch.astype(jnp.int32).reshape(N, 1)
    lin_b2d = lin_b.reshape(1, 1)

    degp = _sc_degree(dst_p, zeros16, ones16, cpt)
    dis, y1 = _tc_dis_y1(degp, x, W1)

    acc1 = _sc_aggregate(y1, src_p, dst_p, zeros128, cpt)
    y2 = _tc_layer(acc1, y1, dis, b1, W2)
    acc2 = _sc_aggregate(y2, src_p, dst_p, zeros128, cpt)
    y3 = _tc_layer(acc2, y2, dis, b2, W3)
    acc3 = _sc_aggregate(y3, src_p, dst_p, zeros128, cpt)
    return _tc_final(acc3, y3, dis, b3, batch2d, lin_W, lin_b2d)


# speedup vs baseline: 10.9993x; 10.9993x over previous
import functools

import jax
import jax.numpy as jnp
from jax import lax
from jax.experimental import pallas as pl
from jax.experimental.pallas import tpu as pltpu
from jax.experimental.pallas import tpu_sc as plsc

N = 10000
D = 128
H = 128
NUM_GRAPHS = 64

NC = 2
NS = 16
NW = NC * NS
CHUNK = 128
ACC = 10112
RPS = ACC // NS

_mesh = lambda: plsc.VectorSubcoreMesh(core_axis_name="c", subcore_axis_name="s")


def _sc_degree(dst_p, zeros128, ones128, cpt):
    @functools.partial(
        pl.kernel,
        out_type=jax.ShapeDtypeStruct((NC, ACC, H), jnp.float32),
        mesh=_mesh(),
        scratch_types=[
            pltpu.VMEM((cpt, CHUNK), jnp.int32),
            pltpu.VMEM((CHUNK, H), jnp.float32),
            pltpu.VMEM_SHARED((ACC, H), jnp.float32),
        ],
    )
    def k(dst_hbm, z_hbm, ones_hbm, out_hbm, dst_v, ones_v, shared):
        c = lax.axis_index("c")
        s = lax.axis_index("s")
        w = c * NS + s
        pltpu.sync_copy(dst_hbm.at[w], dst_v)
        pltpu.sync_copy(ones_hbm, ones_v)
        pltpu.sync_copy(z_hbm.at[pl.ds(s * RPS, RPS)], shared.at[pl.ds(s * RPS, RPS)])
        plsc.subcore_barrier()

        @pl.loop(0, cpt)
        def _(i):
            pltpu.sync_copy(ones_v, shared.at[dst_v.at[i]], add=True)

        plsc.subcore_barrier()
        pltpu.sync_copy(shared.at[pl.ds(s * RPS, RPS)],
                        out_hbm.at[c, pl.ds(s * RPS, RPS)])

    return k(dst_p, zeros128, ones128)


def _sc_aggregate(y, src_p, dst_p, zeros128, cpt):
    @functools.partial(
        pl.kernel,
        out_type=jax.ShapeDtypeStruct((NC, ACC, H), jnp.float32),
        mesh=_mesh(),
        scratch_types=[
            pltpu.VMEM((cpt, CHUNK), jnp.int32),
            pltpu.VMEM((cpt, CHUNK), jnp.int32),
            pltpu.VMEM((CHUNK, H), jnp.float32),
            pltpu.VMEM_SHARED((ACC, H), jnp.float32),
        ],
    )
    def k(y_hbm, src_hbm, dst_hbm, z_hbm, out_hbm, src_v, dst_v, rows_v, shared):
        c = lax.axis_index("c")
        s = lax.axis_index("s")
        w = c * NS + s
        pltpu.sync_copy(src_hbm.at[w], src_v)
        pltpu.sync_copy(dst_hbm.at[w], dst_v)
        pltpu.sync_copy(z_hbm.at[pl.ds(s * RPS, RPS)], shared.at[pl.ds(s * RPS, RPS)])
        plsc.subcore_barrier()

        @pl.loop(0, cpt)
        def _(i):
            pltpu.sync_copy(y_hbm.at[src_v.at[i]], rows_v)
            pltpu.sync_copy(rows_v, shared.at[dst_v.at[i]], add=True)

        plsc.subcore_barrier()
        pltpu.sync_copy(shared.at[pl.ds(s * RPS, RPS)],
                        out_hbm.at[c, pl.ds(s * RPS, RPS)])

    return k(y, src_p, dst_p, zeros128)


def _tc_dis_y1(degp, x, W1):
    def body(deg_ref, x_ref, w_ref, dis_ref, y_ref):
        d = deg_ref[0, :N, 0:1] + deg_ref[1, :N, 0:1]
        dis = lax.rsqrt(d + 1.0)
        dis_ref[...] = dis
        y_ref[...] = jnp.dot(x_ref[...], w_ref[...],
                             preferred_element_type=jnp.float32) * dis

    return pl.pallas_call(
        body,
        out_shape=(jax.ShapeDtypeStruct((N, 1), jnp.float32),
                   jax.ShapeDtypeStruct((N, H), jnp.float32)),
    )(degp, x, W1)


def _tc_layer(acc, y, dis, b, W_next):
    def body(acc_ref, y_ref, dis_ref, b_ref, w_ref, out_ref):
        dis = dis_ref[...]
        h = jnp.maximum((acc_ref[0, :N] + acc_ref[1, :N] + y_ref[...]) * dis
                        + b_ref[...], 0.0)
        out_ref[...] = jnp.dot(h, w_ref[...],
                               preferred_element_type=jnp.float32) * dis

    return pl.pallas_call(
        body,
        out_shape=jax.ShapeDtypeStruct((N, H), jnp.float32),
    )(acc, y, dis, b, W_next)


def _tc_final(acc, y, dis, b, batch2d, lin_W, lin_b2d):
    def body(acc_ref, y_ref, dis_ref, b_ref, batch_ref, lw_ref, lb_ref, out_ref):
        dis = dis_ref[...]
        h = jnp.maximum((acc_ref[0, :N] + acc_ref[1, :N] + y_ref[...]) * dis
                        + b_ref[...], 0.0)
        g = lax.broadcasted_iota(jnp.int32, (1, NUM_GRAPHS), 1)
        m = (batch_ref[...] == g).astype(jnp.float32)
        sums = lax.dot_general(m, h, (((0,), (0,)), ((), ())),
                               preferred_element_type=jnp.float32)
        cnts = jnp.sum(m, axis=0)[:, None]
        pooled = sums / jnp.maximum(cnts, 1.0)
        out_ref[...] = jnp.dot(pooled, lw_ref[...],
                               preferred_element_type=jnp.float32) + lb_ref[...]

    return pl.pallas_call(
        body,
        out_shape=jax.ShapeDtypeStruct((NUM_GRAPHS, 1), jnp.float32),
    )(acc, y, dis, b, batch2d, lin_W, lin_b2d)


def kernel(x, edge_index, batch, W1, b1, W2, b2, W3, b3, lin_W, lin_b):
    E = edge_index.shape[1]
    epad = ((E + NW * CHUNK - 1) // (NW * CHUNK)) * (NW * CHUNK)
    cpt = epad // (NW * CHUNK)

    src = edge_index[0].astype(jnp.int32)
    dst = edge_index[1].astype(jnp.int32)
    src_p = jnp.concatenate(
        [src, jnp.zeros((epad - E,), jnp.int32)]).reshape(NW, cpt, CHUNK)
    dst_p = jnp.concatenate(
        [dst, jnp.full((epad - E,), N, jnp.int32)]).reshape(NW, cpt, CHUNK)

    zeros128 = jnp.zeros((ACC, H), jnp.float32)
    ones128 = jnp.ones((CHUNK, H), jnp.float32)
    batch2d = batch.astype(jnp.int32).reshape(N, 1)
    lin_b2d = lin_b.reshape(1, 1)

    degp = _sc_degree(dst_p, zeros128, ones128, cpt)
    dis, y1 = _tc_dis_y1(degp, x, W1)

    acc1 = _sc_aggregate(y1, src_p, dst_p, zeros128, cpt)
    y2 = _tc_layer(acc1, y1, dis, b1, W2)
    acc2 = _sc_aggregate(y2, src_p, dst_p, zeros128, cpt)
    y3 = _tc_layer(acc2, y2, dis, b2, W3)
    acc3 = _sc_aggregate(y3, src_p, dst_p, zeros128, cpt)
    return _tc_final(acc3, y3, dis, b3, batch2d, lin_W, lin_b2d)
